# baseline (speedup 1.0000x reference)
import jax
import jax.numpy as jnp
from jax import lax
from jax.experimental import pallas as pl
from jax.experimental.pallas import tpu as pltpu
from jax.experimental.pallas import tpu_sc as plsc

N = 25000
E = 100000
IN_FEAT = 16
OUT_FEAT = 16
NUM_RELS = 16
NUM_RULES = 4

NW = 32
EW = 3200
E_PAD = NW * EW
SUB = 128
NSUB = EW // SUB
N_PAD = 25088
ROWS_PER_TILE = N_PAD // 16
XROW = 64
XUSED = NUM_RULES * OUT_FEAT


def _xform_body(f_ref, w_ref, b_ref, o_ref):
    o_ref[...] = (
        jnp.dot(f_ref[...], w_ref[0], preferred_element_type=jnp.float32)
        + b_ref[0]
    ).astype(jnp.bfloat16)


def _compute_x(feat, w4, b4):
    blk = N_PAD
    nb = 1
    return pl.pallas_call(
        _xform_body,
        grid=(nb, NUM_RELS // 2),
        in_specs=[
            pl.BlockSpec((blk, IN_FEAT), lambda i, k2: (i, 0)),
            pl.BlockSpec((1, IN_FEAT, 2 * XUSED), lambda i, k2: (k2, 0, 0)),
            pl.BlockSpec((1, 1, 2 * XUSED), lambda i, k2: (k2, 0, 0)),
        ],
        out_specs=pl.BlockSpec((blk, 2 * XUSED), lambda i, k2: (k2 * nb + i, 0)),
        out_shape=jax.ShapeDtypeStruct((N_PAD * NUM_RELS // 2, 2 * XUSED), jnp.bfloat16),
    )(feat, w4, b4)


def _edge_body(xrows, xidx_w, dst_w, tv_w, out_partial,
               dst_v, tv_v, xidx_v, xbuf0, xbuf1, msg0, msg1,
               accum, sem0, sem1):
    cid = lax.axis_index("c")
    sid = lax.axis_index("s")
    wid = cid * 16 + sid

    zeros16 = jnp.zeros((16,), jnp.float32)

    pltpu.sync_copy(xidx_w.at[wid], xidx_v)
    pltpu.sync_copy(dst_w.at[wid], dst_v)
    pltpu.sync_copy(tv_w.at[wid], tv_v.at[pl.ds(0, EW * NUM_RULES)])

    for i in range(SUB):
        msg0[i, :] = zeros16
    for q in range(ROWS_PER_TILE // SUB):
        pltpu.sync_copy(msg0, accum.at[pl.ds(sid * ROWS_PER_TILE + q * SUB, SUB)])
    rem = ROWS_PER_TILE % SUB
    if rem:
        pltpu.sync_copy(
            msg0.at[pl.ds(0, rem)],
            accum.at[pl.ds(sid * ROWS_PER_TILE + (ROWS_PER_TILE // SUB) * SUB, rem)],
        )

    plsc.subcore_barrier()

    def process(c, xbuf, msg):
        base4 = c * (SUB * NUM_RULES)
        for i in range(SUB):
            tvv = tv_v[pl.ds(base4 + i * NUM_RULES, 16)]
            acc = tvv[0] * xbuf[i, pl.ds(0, 16)].astype(jnp.float32)
            for r in range(1, NUM_RULES):
                acc = acc + tvv[r] * xbuf[i, pl.ds(r * 16, 16)].astype(jnp.float32)
            msg[i, :] = acc
        pltpu.sync_copy(msg, accum.at[dst_v.at[c]], add=True)

    def sub_chunk(c, _):
        pltpu.async_copy(xrows.at[xidx_v.at[c]], xbuf0, sem0).wait()
        process(c, xbuf0, msg0)
        return ()

    lax.fori_loop(0, NSUB, sub_chunk, (), unroll=False)

    plsc.subcore_barrier()

    pltpu.sync_copy(
        accum.at[pl.ds(sid * ROWS_PER_TILE, ROWS_PER_TILE)],
        out_partial.at[cid, pl.ds(sid * ROWS_PER_TILE, ROWS_PER_TILE)],
    )


def _edge_pass(xrows, xidx_w, dst_w, tv_w):
    mesh = plsc.VectorSubcoreMesh(core_axis_name="c", subcore_axis_name="s")
    fn = pl.kernel(
        _edge_body,
        mesh=mesh,
        compiler_params=pltpu.CompilerParams(use_tc_tiling_on_sc=False),
        out_type=jax.ShapeDtypeStruct((2, N_PAD, OUT_FEAT), jnp.float32),
        scratch_types=[
            pltpu.VMEM((NSUB, SUB), jnp.int32),
            pltpu.VMEM((EW * NUM_RULES + 16,), jnp.float32),
            pltpu.VMEM((NSUB, SUB), jnp.int32),
            pltpu.VMEM((SUB, XROW), jnp.bfloat16),
            pltpu.VMEM((SUB, XROW), jnp.bfloat16),
            pltpu.VMEM((SUB, OUT_FEAT), jnp.float32),
            pltpu.VMEM((SUB, OUT_FEAT), jnp.float32),
            pltpu.VMEM_SHARED((N_PAD, OUT_FEAT), jnp.float32),
            pltpu.SemaphoreType.DMA,
            pltpu.SemaphoreType.DMA,
        ],
    )
    return fn(xrows, xidx_w, dst_w, tv_w)


def _sum_body(p_ref, o_ref):
    o_ref[...] = p_ref[0] + p_ref[1]


def _sum_partials(partial):
    return pl.pallas_call(
        _sum_body,
        grid=(16,),
        in_specs=[pl.BlockSpec((2, ROWS_PER_TILE, OUT_FEAT), lambda i: (0, i, 0))],
        out_specs=pl.BlockSpec((ROWS_PER_TILE, OUT_FEAT), lambda i: (i, 0)),
        out_shape=jax.ShapeDtypeStruct((N_PAD, OUT_FEAT), jnp.float32),
    )(partial)


@jax.jit
def kernel(feat, edge_index, etypes, truth_value, weight, h_bias):
    w3 = weight.transpose(0, 2, 1, 3).reshape(NUM_RELS, IN_FEAT, XUSED)
    w4 = (
        w3.reshape(NUM_RELS // 2, 2, IN_FEAT, XUSED)
        .transpose(0, 2, 1, 3)
        .reshape(NUM_RELS // 2, IN_FEAT, 2 * XUSED)
    )
    b4 = h_bias.reshape(NUM_RELS // 2, 1, 2 * XUSED)

    feat_p = jnp.concatenate(
        [feat, jnp.zeros((N_PAD - N, IN_FEAT), jnp.float32)])
    xrows = _compute_x(feat_p, w4, b4).reshape(N_PAD * NUM_RELS, XROW)

    src = edge_index[0]
    dst = edge_index[1]
    xidx = (etypes >> 1) * (2 * N_PAD) + 2 * src + (etypes & 1)
    pad = E_PAD - E
    xidx_p = jnp.concatenate([xidx, jnp.zeros((pad,), jnp.int32)])
    dst_p = jnp.concatenate([dst, jnp.zeros((pad,), jnp.int32)])
    tv_p = jnp.concatenate(
        [truth_value.reshape(E, NUM_RULES),
         jnp.zeros((pad, NUM_RULES), jnp.float32)])

    xidx_w = xidx_p.reshape(NW, NSUB, SUB)
    dst_w = dst_p.reshape(NW, NSUB, SUB)
    tv_w = tv_p.reshape(NW, EW * NUM_RULES)

    partial = _edge_pass(xrows, xidx_w, dst_w, tv_w)
    summed = _sum_partials(partial)
    return summed[:N].reshape(N, 1, OUT_FEAT)


# speedup vs baseline: 1.5256x; 1.5256x over previous
import jax
import jax.numpy as jnp
from jax import lax
from jax.experimental import pallas as pl
from jax.experimental.pallas import tpu as pltpu
from jax.experimental.pallas import tpu_sc as plsc

N = 25000
E = 100000
IN_FEAT = 16
OUT_FEAT = 16
NUM_RELS = 16
NUM_RULES = 4

NW = 32
EW = 3200
E_PAD = NW * EW
SUB = 128
NSUB = EW // SUB
N_PAD = 25088
ROWS_PER_TILE = N_PAD // 16
XROW = 64
XUSED = NUM_RULES * OUT_FEAT


def _xform_body(f_ref, w_ref, b_ref, o_ref):
    o_ref[...] = (
        jnp.dot(f_ref[...], w_ref[0], preferred_element_type=jnp.float32)
        + b_ref[0]
    )


def _compute_x(feat, w4, b4):
    blk = N
    nb = 1
    return pl.pallas_call(
        _xform_body,
        grid=(nb, NUM_RELS // 2),
        in_specs=[
            pl.BlockSpec((blk, IN_FEAT), lambda i, k2: (i, 0)),
            pl.BlockSpec((1, IN_FEAT, 2 * XUSED), lambda i, k2: (k2, 0, 0)),
            pl.BlockSpec((1, 1, 2 * XUSED), lambda i, k2: (k2, 0, 0)),
        ],
        out_specs=pl.BlockSpec((blk, 2 * XUSED), lambda i, k2: (k2 * nb + i, 0)),
        out_shape=jax.ShapeDtypeStruct((N * NUM_RELS // 2, 2 * XUSED), jnp.float32),
    )(feat, w4, b4)


def _edge_body(xrows, xidx_w, dst_w, tv_w, out_partial,
               dst_v, tv_v, xidx_v, xbuf0, xbuf1, msg0, msg1,
               accum, sem0, sem1):
    cid = lax.axis_index("c")
    sid = lax.axis_index("s")
    wid = cid * 16 + sid

    zeros16 = jnp.zeros((16,), jnp.float32)

    pltpu.sync_copy(xidx_w.at[wid], xidx_v)
    pltpu.sync_copy(dst_w.at[wid], dst_v)
    pltpu.sync_copy(tv_w.at[wid], tv_v.at[pl.ds(0, EW * NUM_RULES)])

    for i in range(SUB):
        msg0[i, :] = zeros16
    for q in range(ROWS_PER_TILE // SUB):
        pltpu.sync_copy(msg0, accum.at[pl.ds(sid * ROWS_PER_TILE + q * SUB, SUB)])
    rem = ROWS_PER_TILE % SUB
    if rem:
        pltpu.sync_copy(
            msg0.at[pl.ds(0, rem)],
            accum.at[pl.ds(sid * ROWS_PER_TILE + (ROWS_PER_TILE // SUB) * SUB, rem)],
        )

    plsc.subcore_barrier()

    def process(c, xbuf, msg):
        base4 = c * (SUB * NUM_RULES)
        for i in range(SUB):
            tvv = tv_v[pl.ds(base4 + i * NUM_RULES, 16)]
            acc = tvv[0] * xbuf[i, pl.ds(0, 16)]
            for r in range(1, NUM_RULES):
                acc = acc + tvv[r] * xbuf[i, pl.ds(r * 16, 16)]
            msg[i, :] = acc
        pltpu.sync_copy(msg, accum.at[dst_v.at[c]], add=True)

    cp0 = pltpu.async_copy(xrows.at[xidx_v.at[0]], xbuf0, sem0)

    def pair(cp, _):
        c0 = 2 * cp
        cp1 = pltpu.async_copy(xrows.at[xidx_v.at[c0 + 1]], xbuf1, sem1)
        pltpu.make_async_copy(xrows.at[xidx_v.at[c0]], xbuf0, sem0).wait()
        process(c0, xbuf0, msg0)
        pltpu.async_copy(xrows.at[xidx_v.at[c0 + 2]], xbuf0, sem0)
        cp1.wait()
        process(c0 + 1, xbuf1, msg1)
        return ()

    lax.fori_loop(0, (NSUB - 1) // 2, pair, (), unroll=False)
    pltpu.make_async_copy(xrows.at[xidx_v.at[NSUB - 1]], xbuf0, sem0).wait()
    process(NSUB - 1, xbuf0, msg0)

    plsc.subcore_barrier()

    pltpu.sync_copy(
        accum.at[pl.ds(sid * ROWS_PER_TILE, ROWS_PER_TILE)],
        out_partial.at[cid, pl.ds(sid * ROWS_PER_TILE, ROWS_PER_TILE)],
    )


def _edge_pass(xrows, xidx_w, dst_w, tv_w):
    mesh = plsc.VectorSubcoreMesh(core_axis_name="c", subcore_axis_name="s")
    fn = pl.kernel(
        _edge_body,
        mesh=mesh,
        compiler_params=pltpu.CompilerParams(use_tc_tiling_on_sc=False),
        out_type=jax.ShapeDtypeStruct((2, N_PAD, OUT_FEAT), jnp.float32),
        scratch_types=[
            pltpu.VMEM((NSUB, SUB), jnp.int32),
            pltpu.VMEM((EW * NUM_RULES + 16,), jnp.float32),
            pltpu.VMEM((NSUB, SUB), jnp.int32),
            pltpu.VMEM((SUB, XROW), jnp.float32),
            pltpu.VMEM((SUB, XROW), jnp.float32),
            pltpu.VMEM((SUB, OUT_FEAT), jnp.float32),
            pltpu.VMEM((SUB, OUT_FEAT), jnp.float32),
            pltpu.VMEM_SHARED((N_PAD, OUT_FEAT), jnp.float32),
            pltpu.SemaphoreType.DMA,
            pltpu.SemaphoreType.DMA,
        ],
    )
    return fn(xrows, xidx_w, dst_w, tv_w)


def _sum_body(p_ref, o_ref):
    o_ref[...] = p_ref[0] + p_ref[1]


def _sum_partials(partial):
    return pl.pallas_call(
        _sum_body,
        grid=(16,),
        in_specs=[pl.BlockSpec((2, ROWS_PER_TILE, OUT_FEAT), lambda i: (0, i, 0))],
        out_specs=pl.BlockSpec((ROWS_PER_TILE, OUT_FEAT), lambda i: (i, 0)),
        out_shape=jax.ShapeDtypeStruct((N_PAD, OUT_FEAT), jnp.float32),
    )(partial)


@jax.jit
def kernel(feat, edge_index, etypes, truth_value, weight, h_bias):
    w3 = weight.transpose(0, 2, 1, 3).reshape(NUM_RELS, IN_FEAT, XUSED)
    w4 = (
        w3.reshape(NUM_RELS // 2, 2, IN_FEAT, XUSED)
        .transpose(0, 2, 1, 3)
        .reshape(NUM_RELS // 2, IN_FEAT, 2 * XUSED)
    )
    b4 = h_bias.reshape(NUM_RELS // 2, 1, 2 * XUSED)

    xrows = _compute_x(feat, w4, b4).reshape(N * NUM_RELS, XROW)

    src = edge_index[0]
    dst = edge_index[1]
    xidx = (etypes >> 1) * (2 * N) + 2 * src + (etypes & 1)
    pad = E_PAD - E
    xidx_p = jnp.concatenate([xidx, jnp.zeros((pad,), jnp.int32)])
    dst_p = jnp.concatenate([dst, jnp.zeros((pad,), jnp.int32)])
    tv_p = jnp.concatenate(
        [truth_value.reshape(E, NUM_RULES),
         jnp.zeros((pad, NUM_RULES), jnp.float32)])

    xidx_w = xidx_p.reshape(NW, NSUB, SUB)
    dst_w = dst_p.reshape(NW, NSUB, SUB)
    tv_w = tv_p.reshape(NW, EW * NUM_RULES)

    partial = _edge_pass(xrows, xidx_w, dst_w, tv_w)
    summed = _sum_partials(partial)
    return summed[:N].reshape(N, 1, OUT_FEAT)


# speedup vs baseline: 1.5260x; 1.0003x over previous
import jax
import jax.numpy as jnp
from jax import lax
from jax.experimental import pallas as pl
from jax.experimental.pallas import tpu as pltpu
from jax.experimental.pallas import tpu_sc as plsc

N = 25000
E = 100000
IN_FEAT = 16
OUT_FEAT = 16
NUM_RELS = 16
NUM_RULES = 4

NW = 32
EW = 3200
E_PAD = NW * EW
SUB = 128
NSUB = EW // SUB
N_PAD = 25088
ROWS_PER_TILE = N_PAD // 16
XROW = 64
XUSED = NUM_RULES * OUT_FEAT


def _xform_body(f_ref, w_ref, b_ref, o_ref):
    o_ref[...] = (
        jnp.dot(f_ref[...], w_ref[0], preferred_element_type=jnp.float32)
        + b_ref[0]
    )


def _compute_x(feat, w4, b4):
    blk = N
    nb = 1
    return pl.pallas_call(
        _xform_body,
        grid=(nb, NUM_RELS // 2),
        compiler_params=pltpu.CompilerParams(
            dimension_semantics=("parallel", "parallel")),
        in_specs=[
            pl.BlockSpec((blk, IN_FEAT), lambda i, k2: (i, 0)),
            pl.BlockSpec((1, IN_FEAT, 2 * XUSED), lambda i, k2: (k2, 0, 0)),
            pl.BlockSpec((1, 1, 2 * XUSED), lambda i, k2: (k2, 0, 0)),
        ],
        out_specs=pl.BlockSpec((blk, 2 * XUSED), lambda i, k2: (k2 * nb + i, 0)),
        out_shape=jax.ShapeDtypeStruct((N * NUM_RELS // 2, 2 * XUSED), jnp.float32),
    )(feat, w4, b4)


def _edge_body(xrows, xidx_w, dst_w, tv_w, out_partial,
               dst_v, tv_v, xidx_v, xbuf0, xbuf1, msg0, msg1,
               accum, sem0, sem1):
    cid = lax.axis_index("c")
    sid = lax.axis_index("s")
    wid = cid * 16 + sid

    zeros16 = jnp.zeros((16,), jnp.float32)

    pltpu.sync_copy(xidx_w.at[wid], xidx_v)
    pltpu.sync_copy(dst_w.at[wid], dst_v)
    pltpu.sync_copy(tv_w.at[wid], tv_v.at[pl.ds(0, EW * NUM_RULES)])

    for i in range(SUB):
        msg0[i, :] = zeros16
    for q in range(ROWS_PER_TILE // SUB):
        pltpu.sync_copy(msg0, accum.at[pl.ds(sid * ROWS_PER_TILE + q * SUB, SUB)])
    rem = ROWS_PER_TILE % SUB
    if rem:
        pltpu.sync_copy(
            msg0.at[pl.ds(0, rem)],
            accum.at[pl.ds(sid * ROWS_PER_TILE + (ROWS_PER_TILE // SUB) * SUB, rem)],
        )

    plsc.subcore_barrier()

    def process(c, xbuf, msg):
        base4 = c * (SUB * NUM_RULES)
        for i in range(SUB):
            tvv = tv_v[pl.ds(base4 + i * NUM_RULES, 16)]
            acc = tvv[0] * xbuf[i, pl.ds(0, 16)]
            for r in range(1, NUM_RULES):
                acc = acc + tvv[r] * xbuf[i, pl.ds(r * 16, 16)]
            msg[i, :] = acc
        pltpu.sync_copy(msg, accum.at[dst_v.at[c]], add=True)

    cp0 = pltpu.async_copy(xrows.at[xidx_v.at[0]], xbuf0, sem0)

    def pair(cp, _):
        c0 = 2 * cp
        cp1 = pltpu.async_copy(xrows.at[xidx_v.at[c0 + 1]], xbuf1, sem1)
        pltpu.make_async_copy(xrows.at[xidx_v.at[c0]], xbuf0, sem0).wait()
        process(c0, xbuf0, msg0)
        pltpu.async_copy(xrows.at[xidx_v.at[c0 + 2]], xbuf0, sem0)
        cp1.wait()
        process(c0 + 1, xbuf1, msg1)
        return ()

    lax.fori_loop(0, (NSUB - 1) // 2, pair, (), unroll=False)
    pltpu.make_async_copy(xrows.at[xidx_v.at[NSUB - 1]], xbuf0, sem0).wait()
    process(NSUB - 1, xbuf0, msg0)

    plsc.subcore_barrier()

    pltpu.sync_copy(
        accum.at[pl.ds(sid * ROWS_PER_TILE, ROWS_PER_TILE)],
        out_partial.at[cid, pl.ds(sid * ROWS_PER_TILE, ROWS_PER_TILE)],
    )


def _edge_pass(xrows, xidx_w, dst_w, tv_w):
    mesh = plsc.VectorSubcoreMesh(core_axis_name="c", subcore_axis_name="s")
    fn = pl.kernel(
        _edge_body,
        mesh=mesh,
        compiler_params=pltpu.CompilerParams(use_tc_tiling_on_sc=False),
        out_type=jax.ShapeDtypeStruct((2, N_PAD, OUT_FEAT), jnp.float32),
        scratch_types=[
            pltpu.VMEM((NSUB, SUB), jnp.int32),
            pltpu.VMEM((EW * NUM_RULES + 16,), jnp.float32),
            pltpu.VMEM((NSUB, SUB), jnp.int32),
            pltpu.VMEM((SUB, XROW), jnp.float32),
            pltpu.VMEM((SUB, XROW), jnp.float32),
            pltpu.VMEM((SUB, OUT_FEAT), jnp.float32),
            pltpu.VMEM((SUB, OUT_FEAT), jnp.float32),
            pltpu.VMEM_SHARED((N_PAD, OUT_FEAT), jnp.float32),
            pltpu.SemaphoreType.DMA,
            pltpu.SemaphoreType.DMA,
        ],
    )
    return fn(xrows, xidx_w, dst_w, tv_w)


def _sum_body(p_ref, o_ref):
    o_ref[...] = p_ref[0] + p_ref[1]


def _sum_partials(partial):
    return pl.pallas_call(
        _sum_body,
        grid=(16,),
        in_specs=[pl.BlockSpec((2, ROWS_PER_TILE, OUT_FEAT), lambda i: (0, i, 0))],
        out_specs=pl.BlockSpec((ROWS_PER_TILE, OUT_FEAT), lambda i: (i, 0)),
        out_shape=jax.ShapeDtypeStruct((N_PAD, OUT_FEAT), jnp.float32),
    )(partial)


@jax.jit
def kernel(feat, edge_index, etypes, truth_value, weight, h_bias):
    w3 = weight.transpose(0, 2, 1, 3).reshape(NUM_RELS, IN_FEAT, XUSED)
    w4 = (
        w3.reshape(NUM_RELS // 2, 2, IN_FEAT, XUSED)
        .transpose(0, 2, 1, 3)
        .reshape(NUM_RELS // 2, IN_FEAT, 2 * XUSED)
    )
    b4 = h_bias.reshape(NUM_RELS // 2, 1, 2 * XUSED)

    xrows = _compute_x(feat, w4, b4).reshape(N * NUM_RELS, XROW)

    src = edge_index[0]
    dst = edge_index[1]
    xidx = (etypes >> 1) * (2 * N) + 2 * src + (etypes & 1)
    pad = E_PAD - E
    xidx_p = jnp.concatenate([xidx, jnp.zeros((pad,), jnp.int32)])
    dst_p = jnp.concatenate([dst, jnp.zeros((pad,), jnp.int32)])
    tv_p = jnp.concatenate(
        [truth_value.reshape(E, NUM_RULES),
         jnp.zeros((pad, NUM_RULES), jnp.float32)])

    xidx_w = xidx_p.reshape(NW, NSUB, SUB)
    dst_w = dst_p.reshape(NW, NSUB, SUB)
    tv_w = tv_p.reshape(NW, EW * NUM_RULES)

    partial = _edge_pass(xrows, xidx_w, dst_w, tv_w)
    summed = _sum_partials(partial)
    return summed[:N].reshape(N, 1, OUT_FEAT)
